# baseline (speedup 1.0000x reference)
import jax
import jax.numpy as jnp
from jax.experimental import pallas as pl

N_USER = 50000
N_ITEM = 50000


def _segment_softmax(logits, seg, num_segments):
    m = jax.ops.segment_max(logits, seg, num_segments=num_segments)
    m = jnp.where(jnp.isfinite(m), m, 0.0)
    e = jnp.exp(logits - m[seg])
    s = jax.ops.segment_sum(e, seg, num_segments=num_segments)
    return e / (s[seg] + 1e-16)


def _gat(x_src, x_dst, edge_index, Ws, Wd, a_s, a_d, b, n_dst):
    hs = x_src @ Ws
    hd = x_dst @ Wd
    src = edge_index[0]
    dst = edge_index[1]
    alpha_src = jnp.take(jnp.sum(hs * a_s, axis=-1), src)
    alpha_dst = jnp.take(jnp.sum(hd * a_d, axis=-1), dst)
    alpha = jax.nn.leaky_relu(alpha_src + alpha_dst, negative_slope=0.2)
    alpha = _segment_softmax(alpha, dst, n_dst)
    msg = alpha[:, None] * jnp.take(hs, src, axis=0)
    out = jax.ops.segment_sum(msg, dst, num_segments=n_dst)
    return out + b


def _pcopy(x):
    return pl.pallas_call(
        lambda x_ref, o_ref: o_ref.__setitem__((...,), x_ref[...]),
        out_shape=jax.ShapeDtypeStruct(x.shape, x.dtype),
    )(x)


def kernel(x_user, x_item, l1_u2i_Ws, l1_u2i_Wd, l1_u2i_as, l1_u2i_ad, l1_u2i_b, l1_i2u_Ws, l1_i2u_Wd, l1_i2u_as, l1_i2u_ad, l1_i2u_b, l2_u2i_Ws, l2_u2i_Wd, l2_u2i_as, l2_u2i_ad, l2_u2i_b, l2_i2u_Ws, l2_i2u_Wd, l2_i2u_as, l2_i2u_ad, l2_i2u_b, rel_weight, edge_index_u2i, edge_index_i2u):
    h_item = _gat(x_user, x_item, edge_index_u2i, l1_u2i_Ws, l1_u2i_Wd, l1_u2i_as, l1_u2i_ad, l1_u2i_b, N_ITEM)
    h_user = _gat(x_item, x_user, edge_index_i2u, l1_i2u_Ws, l1_i2u_Wd, l1_i2u_as, l1_i2u_ad, l1_i2u_b, N_USER)
    h_item = jax.nn.relu(h_item)
    h_user = jax.nn.relu(h_user)
    o_item = _gat(h_user, h_item, edge_index_u2i, l2_u2i_Ws, l2_u2i_Wd, l2_u2i_as, l2_u2i_ad, l2_u2i_b, N_ITEM)
    o_user = _gat(h_item, h_user, edge_index_i2u, l2_i2u_Ws, l2_i2u_Wd, l2_i2u_as, l2_i2u_ad, l2_i2u_b, N_USER)
    pred_u2i = jnp.sum(jnp.take(o_user, edge_index_u2i[0], axis=0) * rel_weight[0] * jnp.take(o_item, edge_index_u2i[1], axis=0), axis=-1)
    pred_i2u = jnp.sum(jnp.take(o_item, edge_index_i2u[0], axis=0) * rel_weight[1] * jnp.take(o_user, edge_index_i2u[1], axis=0), axis=-1)
    return _pcopy(jnp.stack([pred_u2i, pred_i2u], axis=0))


# speedup vs baseline: 6.1528x; 6.1528x over previous
import functools

import jax
import jax.numpy as jnp
from jax import lax
from jax.experimental import pallas as pl
from jax.experimental.pallas import tpu as pltpu
from jax.experimental.pallas import tpu_sc as plsc

N = 50000
D = 128
E = 400000
MBLK = 512
NPAD = 50176
NC = 2
NS = 16
NW = NC * NS
EPW = 12800
EPAD = NW * EPW
NSLAB = 8
SLAB = D // NSLAB
BLK = 128
NBLKS = EPW // BLK
EPWE = EPAD // NS
CHE = 2560
NCHE = EPWE // CHE
CBLKS = CHE // BLK
ROWS_PT = NPAD // NS


def _dense_tc(xs_ref, xd_ref, ws_ref, wd_ref, as_ref, ad_ref, *out_refs):
    hs_refs = out_refs[:NSLAB]
    ssrc_ref, sdst_ref = out_refs[NSLAB], out_refs[NSLAB + 1]
    xs = xs_ref[...]
    hs = jnp.dot(xs, ws_ref[...], preferred_element_type=jnp.float32)
    for q in range(NSLAB):
        hs_refs[q][...] = hs[:, q * SLAB:(q + 1) * SLAB]
    ssrc_ref[...] = jnp.dot(hs, as_ref[...], preferred_element_type=jnp.float32)
    wd = jnp.dot(wd_ref[...], ad_ref[...], preferred_element_type=jnp.float32)
    sdst_ref[...] = jnp.dot(xd_ref[...], wd, preferred_element_type=jnp.float32)


def _dense(x_src, x_dst, Ws, Wd, a_s, a_d):
    grid = NPAD // MBLK
    outs = pl.pallas_call(
        _dense_tc,
        grid=(grid,),
        in_specs=[
            pl.BlockSpec((MBLK, D), lambda i: (i, 0)),
            pl.BlockSpec((MBLK, D), lambda i: (i, 0)),
            pl.BlockSpec((D, D), lambda i: (0, 0)),
            pl.BlockSpec((D, D), lambda i: (0, 0)),
            pl.BlockSpec((D, 1), lambda i: (0, 0)),
            pl.BlockSpec((D, 1), lambda i: (0, 0)),
        ],
        out_specs=[pl.BlockSpec((MBLK, SLAB), lambda i: (i, 0))] * NSLAB
        + [pl.BlockSpec((MBLK, 1), lambda i: (i, 0))] * 2,
        out_shape=[jax.ShapeDtypeStruct((NPAD, SLAB), jnp.float32)] * NSLAB
        + [jax.ShapeDtypeStruct((NPAD, 1), jnp.float32)] * 2,
    )(x_src, x_dst, Ws, Wd, a_s.reshape(D, 1), a_d.reshape(D, 1))
    return outs[:NSLAB], outs[NSLAB].reshape(NPAD), outs[NSLAB + 1].reshape(NPAD)


def _finalize_tc(relu, acc8_ref, ssum_ref, b_ref, w_ref, o_ref, obf_ref,
                 owbf_ref):
    i = pl.program_id(0)
    acc = jnp.concatenate([acc8_ref[q] for q in range(NSLAB)], axis=1)
    s = ssum_ref[...]
    o = acc / (s + 1e-16) + b_ref[...]
    if relu:
        o = jnp.maximum(o, 0.0)
    row = i * MBLK + lax.broadcasted_iota(jnp.int32, (MBLK, 1), 0)
    o = jnp.where(row < N, o, 0.0)
    o_ref[...] = o
    obf_ref[...] = o.astype(jnp.bfloat16)
    owbf_ref[...] = (o * w_ref[...]).astype(jnp.bfloat16)


def _finalize(acc8, ssum, b, w, relu):
    grid = NPAD // MBLK
    o, obf, owbf = pl.pallas_call(
        functools.partial(_finalize_tc, relu),
        grid=(grid,),
        in_specs=[
            pl.BlockSpec((NSLAB, MBLK, SLAB), lambda i: (0, i, 0)),
            pl.BlockSpec((MBLK, 1), lambda i: (i, 0)),
            pl.BlockSpec((1, D), lambda i: (0, 0)),
            pl.BlockSpec((1, D), lambda i: (0, 0)),
        ],
        out_specs=[
            pl.BlockSpec((MBLK, D), lambda i: (i, 0)),
            pl.BlockSpec((MBLK, D), lambda i: (i, 0)),
            pl.BlockSpec((MBLK, D), lambda i: (i, 0)),
        ],
        out_shape=[
            jax.ShapeDtypeStruct((NPAD, D), jnp.float32),
            jax.ShapeDtypeStruct((NPAD, D), jnp.bfloat16),
            jax.ShapeDtypeStruct((NPAD, D), jnp.bfloat16),
        ],
    )(acc8, ssum.reshape(NPAD, 1), b.reshape(1, D), w.reshape(1, D))
    return o, obf, owbf


def _dot_tc(ga_ref, gb_ref, p_ref):
    a = ga_ref[...].astype(jnp.float32)
    b = gb_ref[...].astype(jnp.float32)
    p_ref[...] = jnp.sum(a * b, axis=2)


def _dot(ga, gb):
    dgrid = EPAD // 2048
    return pl.pallas_call(
        _dot_tc,
        grid=(dgrid,),
        in_specs=[
            pl.BlockSpec((2, 2048, D), lambda i: (0, i, 0)),
            pl.BlockSpec((2, 2048, D), lambda i: (0, i, 0)),
        ],
        out_specs=pl.BlockSpec((2, 2048), lambda i: (0, i)),
        out_shape=jax.ShapeDtypeStruct((2, EPAD), jnp.float32),
    )(ga, gb)


def _edge_body(h0, h1, h2, h3, h4, h5, h6, h7, ssrc_hbm, sdst_hbm,
               src_hbm, dst_hbm, acc8_hbm, ssum_hbm, w_hbm,
               src_v, dst_v, wv_v, sidx_v, ablk_v, bblk_v, rows_v, zrow_v,
               zsum_v, acc_sh, ssum_sh,
               sem_r0, sem_r1, sem_s0, sem_s1, sem_a, sem_b, sem_w):
    hq = (h0, h1, h2, h3, h4, h5, h6, h7)
    cid = lax.axis_index("c")
    sid = lax.axis_index("s")
    ebase = sid * EPWE
    zf = jnp.zeros((16,), jnp.float32)
    sem_r = (sem_r0, sem_r1)
    sem_s = (sem_s0, sem_s1)

    def zr_body(r, _):
        zrow_v[r, pl.ds(0, SLAB)] = zf
        return 0
    lax.fori_loop(0, BLK, zr_body, 0)

    def zs_body(i, _):
        zsum_v[pl.ds(i * 16, 16)] = zf
        return 0
    lax.fori_loop(0, ROWS_PT // 16, zs_body, 0)

    row0 = sid * ROWS_PT

    def stage_sidx(u, j):
        for g in range(BLK // 16):
            sidx_v[u, pl.ds(g * 16, 16)] = dst_v[pl.ds(j * BLK + g * 16, 16)]

    def g_desc(u, j, q):
        return pltpu.make_async_copy(
            hq[q].at[src_v.at[pl.ds(j * BLK, BLK)]], rows_v.at[u], sem_r[u])

    def s_desc(u):
        return pltpu.make_async_copy(
            rows_v.at[u], acc_sh.at[sidx_v.at[u]], sem_s[u])

    def fire_scatter(u):
        pltpu.async_copy(rows_v.at[u], acc_sh.at[sidx_v.at[u]], sem_s[u],
                         add=True)

    def fire_gather(u, j, q):
        pltpu.async_copy(hq[q].at[src_v.at[pl.ds(j * BLK, BLK)]],
                         rows_v.at[u], sem_r[u])

    def compute_w(j):
        csl = pl.ds(j * BLK, BLK)
        acp = pltpu.async_copy(ssrc_hbm.at[src_v.at[csl]], ablk_v, sem_a)
        bcp = pltpu.async_copy(sdst_hbm.at[dst_v.at[csl]], bblk_v, sem_b)
        acp.wait()
        bcp.wait()
        for g in range(BLK // 16):
            a = ablk_v[pl.ds(g * 16, 16)]
            bb = bblk_v[pl.ds(g * 16, 16)]
            x = a + bb
            x = jnp.where(x >= 0.0, x, 0.2 * x)
            wv_v[pl.ds(j * BLK + g * 16, 16)] = jnp.exp(x)

    def scale(u, j):
        def scale_body(g, _):
            wvec = wv_v[pl.ds(j * BLK + g * 16, 16)]
            for l in range(16):
                e = g * 16 + l
                rows_v[u, e, pl.ds(0, SLAB)] = \
                    rows_v[u, e, pl.ds(0, SLAB)] * wvec[l]
            return 0
        lax.fori_loop(0, BLK // 16, scale_body, 0)

    for q in range(NSLAB):
        p = q % 4
        @pl.when(q // 4 == cid)
        def _(q=q, p=p):
            for t in range(ROWS_PT // BLK):
                pltpu.sync_copy(zrow_v, acc_sh.at[pl.ds(row0 + t * BLK, BLK)])
            rem = ROWS_PT % BLK
            if rem:
                pltpu.sync_copy(zrow_v.at[pl.ds(0, rem)],
                                acc_sh.at[pl.ds(row0 + (ROWS_PT // BLK) * BLK,
                                                rem)])
            if q == 0:
                pltpu.sync_copy(zsum_v, ssum_sh.at[pl.ds(row0, ROWS_PT)])
            plsc.subcore_barrier()

            def chunk_body(ch, _):
                cb = ebase + ch * CHE
                pltpu.sync_copy(src_hbm.at[pl.ds(cb, CHE)], src_v)
                pltpu.sync_copy(dst_hbm.at[pl.ds(cb, CHE)], dst_v)
                if p != 0:
                    pltpu.sync_copy(w_hbm.at[cid, pl.ds(cb, CHE)], wv_v)

                stage_sidx(0, 0)
                if p == 0:
                    compute_w(0)
                fire_gather(0, 0, q)

                def pair_body(t, _):
                    for u in (0, 1):
                        j = 2 * t + u
                        un = 1 - u
                        @pl.when(j >= 1)
                        def _():
                            s_desc(un).wait()
                        @pl.when(j + 1 < CBLKS)
                        def _():
                            stage_sidx(un, j + 1)
                            if p == 0:
                                compute_w(j + 1)
                            fire_gather(un, j + 1, q)
                        g_desc(u, j, q).wait()
                        scale(u, j)
                        fire_scatter(u)
                        if q == 0:
                            pltpu.async_copy(wv_v.at[pl.ds(j * BLK, BLK)],
                                             ssum_sh.at[sidx_v.at[u]], sem_w,
                                             add=True).wait()
                    return 0
                lax.fori_loop(0, CBLKS // 2, pair_body, 0)
                s_desc((CBLKS - 1) % 2).wait()
                if p == 0:
                    pltpu.sync_copy(wv_v, w_hbm.at[cid, pl.ds(cb, CHE)])
                return 0
            lax.fori_loop(0, NCHE, chunk_body, 0)

            plsc.subcore_barrier()
            pltpu.sync_copy(acc_sh.at[pl.ds(row0, ROWS_PT)],
                            acc8_hbm.at[q, pl.ds(row0, ROWS_PT)])
            if q == 0:
                pltpu.sync_copy(ssum_sh.at[pl.ds(row0, ROWS_PT)],
                                ssum_hbm.at[pl.ds(row0, ROWS_PT)])
            plsc.subcore_barrier()


def _edge(hs_slabs, ssrc, sdst, src_p, dst_p):
    mesh = plsc.VectorSubcoreMesh(core_axis_name="c", subcore_axis_name="s")
    kern = pl.kernel(
        _edge_body,
        out_type=(jax.ShapeDtypeStruct((NSLAB, NPAD, SLAB), jnp.float32),
                  jax.ShapeDtypeStruct((NPAD,), jnp.float32),
                  jax.ShapeDtypeStruct((NC, EPAD), jnp.float32)),
        mesh=mesh,
        compiler_params=pltpu.CompilerParams(use_tc_tiling_on_sc=False),
        scratch_types=[
            pltpu.VMEM((CHE,), jnp.int32),
            pltpu.VMEM((CHE,), jnp.int32),
            pltpu.VMEM((CHE,), jnp.float32),
            pltpu.VMEM((2, BLK), jnp.int32),
            pltpu.VMEM((BLK,), jnp.float32),
            pltpu.VMEM((BLK,), jnp.float32),
            pltpu.VMEM((2, BLK, SLAB), jnp.float32),
            pltpu.VMEM((BLK, SLAB), jnp.float32),
            pltpu.VMEM((ROWS_PT,), jnp.float32),
            pltpu.VMEM_SHARED((NPAD, SLAB), jnp.float32),
            pltpu.VMEM_SHARED((NPAD,), jnp.float32),
            pltpu.SemaphoreType.DMA,
            pltpu.SemaphoreType.DMA,
            pltpu.SemaphoreType.DMA,
            pltpu.SemaphoreType.DMA,
            pltpu.SemaphoreType.DMA,
            pltpu.SemaphoreType.DMA,
            pltpu.SemaphoreType.DMA,
        ],
    )
    acc8, ssum, _unused_w = kern(*hs_slabs, ssrc, sdst, src_p, dst_p)
    return acc8, ssum


def _pgather_body(a0_hbm, b0_hbm, a1_hbm, b1_hbm, s0_hbm, d0_hbm,
                  s1_hbm, d1_hbm, ga_hbm, gb_hbm,
                  ps_v, pd_v, arow_v, brow_v,
                  sem_a0, sem_a1, sem_b0, sem_b1, sem_o0, sem_o1):
    cid = lax.axis_index("c")
    sid = lax.axis_index("s")
    wid = cid * NS + sid
    ebase = wid * EPW
    sem_a = (sem_a0, sem_a1)
    sem_b = (sem_b0, sem_b1)
    sem_o = (sem_o0, sem_o1)

    for r in range(2):
        ah = a0_hbm if r == 0 else a1_hbm
        bh = b0_hbm if r == 0 else b1_hbm
        sh = s0_hbm if r == 0 else s1_hbm
        dh = d0_hbm if r == 0 else d1_hbm
        pltpu.sync_copy(sh.at[pl.ds(ebase, EPW)], ps_v)
        pltpu.sync_copy(dh.at[pl.ds(ebase, EPW)], pd_v)

        def fire_g(u, j):
            csl = pl.ds(j * BLK, BLK)
            pltpu.async_copy(ah.at[ps_v.at[csl]], arow_v.at[u], sem_a[u])
            pltpu.async_copy(bh.at[pd_v.at[csl]], brow_v.at[u], sem_b[u])

        def wait_g(u, j):
            csl = pl.ds(j * BLK, BLK)
            pltpu.make_async_copy(ah.at[ps_v.at[csl]], arow_v.at[u],
                                  sem_a[u]).wait()
            pltpu.make_async_copy(bh.at[pd_v.at[csl]], brow_v.at[u],
                                  sem_b[u]).wait()

        def fire_o(u, j):
            osl = pl.ds(ebase + j * BLK, BLK)
            pltpu.async_copy(arow_v.at[u], ga_hbm.at[r, osl], sem_o[u])
            pltpu.async_copy(brow_v.at[u], gb_hbm.at[r, osl], sem_o[u])

        def wait_o(u, j):
            osl = pl.ds(ebase + j * BLK, BLK)
            pltpu.make_async_copy(arow_v.at[u], ga_hbm.at[r, osl],
                                  sem_o[u]).wait()
            pltpu.make_async_copy(brow_v.at[u], gb_hbm.at[r, osl],
                                  sem_o[u]).wait()

        fire_g(0, 0)

        def pair_body(t, _):
            for u in (0, 1):
                j = 2 * t + u
                un = 1 - u
                @pl.when(j >= 1)
                def _():
                    wait_o(un, j - 1)
                @pl.when(j + 1 < NBLKS)
                def _():
                    fire_g(un, j + 1)
                wait_g(u, j)
                fire_o(u, j)
            return 0
        lax.fori_loop(0, NBLKS // 2, pair_body, 0)
        wait_o((NBLKS - 1) % 2, NBLKS - 1)


def _pgather(a0, b0, a1, b1, s0, d0, s1, d1):
    mesh = plsc.VectorSubcoreMesh(core_axis_name="c", subcore_axis_name="s")
    kern = pl.kernel(
        _pgather_body,
        out_type=(jax.ShapeDtypeStruct((2, EPAD, D // 2), jnp.int32),
                  jax.ShapeDtypeStruct((2, EPAD, D // 2), jnp.int32)),
        mesh=mesh,
        compiler_params=pltpu.CompilerParams(use_tc_tiling_on_sc=False),
        scratch_types=[
            pltpu.VMEM((EPW,), jnp.int32),
            pltpu.VMEM((EPW,), jnp.int32),
            pltpu.VMEM((2, BLK, D // 2), jnp.int32),
            pltpu.VMEM((2, BLK, D // 2), jnp.int32),
            pltpu.SemaphoreType.DMA,
            pltpu.SemaphoreType.DMA,
            pltpu.SemaphoreType.DMA,
            pltpu.SemaphoreType.DMA,
            pltpu.SemaphoreType.DMA,
            pltpu.SemaphoreType.DMA,
        ],
    )
    return kern(a0, b0, a1, b1, s0, d0, s1, d1)


def kernel(x_user, x_item, l1_u2i_Ws, l1_u2i_Wd, l1_u2i_as, l1_u2i_ad,
           l1_u2i_b, l1_i2u_Ws, l1_i2u_Wd, l1_i2u_as, l1_i2u_ad, l1_i2u_b,
           l2_u2i_Ws, l2_u2i_Wd, l2_u2i_as, l2_u2i_ad, l2_u2i_b,
           l2_i2u_Ws, l2_i2u_Wd, l2_i2u_as, l2_i2u_ad, l2_i2u_b,
           rel_weight, edge_index_u2i, edge_index_i2u):
    f32 = jnp.float32
    npe = EPAD - E
    pad_src = (jnp.arange(npe, dtype=jnp.int32) * 131) % N
    pad_dst = N + (jnp.arange(npe, dtype=jnp.int32) % (NPAD - N))

    def prep(ei):
        s = jnp.concatenate([ei[0].astype(jnp.int32), pad_src])
        d = jnp.concatenate([ei[1].astype(jnp.int32), pad_dst])
        return s, d

    s0_p, d0_p = prep(edge_index_u2i)
    s1_p, d1_p = prep(edge_index_i2u)

    xu = jnp.pad(x_user.astype(f32), ((0, NPAD - N), (0, 0)))
    xi = jnp.pad(x_item.astype(f32), ((0, NPAD - N), (0, 0)))

    hsl0, ssrc0, sdst0 = _dense(xu, xi, l1_u2i_Ws, l1_u2i_Wd, l1_u2i_as,
                                l1_u2i_ad)
    hsl1, ssrc1, sdst1 = _dense(xi, xu, l1_i2u_Ws, l1_i2u_Wd, l1_i2u_as,
                                l1_i2u_ad)
    acc8_i, ssum_i = _edge(hsl0, ssrc0, sdst0, s0_p, d0_p)
    acc8_u, ssum_u = _edge(hsl1, ssrc1, sdst1, s1_p, d1_p)
    ones = jnp.ones((D,), f32)
    h_item, _, _ = _finalize(acc8_i, ssum_i, l1_u2i_b, ones, True)
    h_user, _, _ = _finalize(acc8_u, ssum_u, l1_i2u_b, ones, True)

    hsl0b, ssrc0b, sdst0b = _dense(h_user, h_item, l2_u2i_Ws, l2_u2i_Wd,
                                   l2_u2i_as, l2_u2i_ad)
    hsl1b, ssrc1b, sdst1b = _dense(h_item, h_user, l2_i2u_Ws, l2_i2u_Wd,
                                   l2_i2u_as, l2_i2u_ad)
    acc8_i2, ssum_i2 = _edge(hsl0b, ssrc0b, sdst0b, s0_p, d0_p)
    acc8_u2, ssum_u2 = _edge(hsl1b, ssrc1b, sdst1b, s1_p, d1_p)
    _, oi_bf, oi_w1 = _finalize(acc8_i2, ssum_i2, l2_u2i_b, rel_weight[1],
                                False)
    _, ou_bf, ou_w0 = _finalize(acc8_u2, ssum_u2, l2_i2u_b, rel_weight[0],
                                False)

    def as_i32(x):
        return lax.bitcast_convert_type(x.reshape(NPAD, D // 2, 2),
                                        jnp.int32)

    ga, gb = _pgather(as_i32(ou_w0), as_i32(oi_bf), as_i32(oi_w1),
                      as_i32(ou_bf), s0_p, d0_p, s1_p, d1_p)

    def as_bf(x):
        return lax.bitcast_convert_type(x, jnp.bfloat16).reshape(2, EPAD, D)

    pred = _dot(as_bf(ga), as_bf(gb))
    return pred[:, :E]
